# Initial kernel scaffold; baseline (speedup 1.0000x reference)
import jax
import jax.numpy as jnp
from jax.experimental import pallas as pl


def kernel(x, emb, W, b):
    raise NotImplementedError("write your pallas kernel here")


# speedup vs baseline: 10.1989x; 10.1989x over previous
import functools

import jax
import jax.numpy as jnp
from jax import lax
from jax.experimental import pallas as pl
from jax.experimental.pallas import tpu as pltpu
from jax.experimental.pallas import tpu_sc as plsc


def _make_gather(V, D, N):
    info = plsc.get_sparse_core_info()
    NC, NS = info.num_cores, info.num_subcores
    NW = NC * NS
    SB = 128
    per_w = N // NW
    assert N % (NW * SB) == 0
    rows_per_w = per_w // SB
    K = 8
    assert rows_per_w % K == 0 and (per_w // SB) % K == 0
    n_chunks = rows_per_w // K
    CH = K * SB

    mesh = plsc.VectorSubcoreMesh(core_axis_name="c", subcore_axis_name="s")

    @functools.partial(
        pl.kernel,
        mesh=mesh,
        out_type=jax.ShapeDtypeStruct((N, D), jnp.float32),
        scratch_types=[
            pltpu.VMEM((K, SB), jnp.int32),
            pltpu.VMEM((CH, D), jnp.float32),
            pltpu.SemaphoreType.DMA,
        ],
        compiler_params=pltpu.CompilerParams(use_tc_tiling_on_sc=False),
    )
    def gather_k(table_hbm, idx_hbm, out_hbm, idx_v, rows_v, sem):
        wid = lax.axis_index("s") * NC + lax.axis_index("c")

        def body(c, carry):
            row0 = wid * rows_per_w + c * K
            pltpu.sync_copy(idx_hbm.at[pl.ds(row0, K)], idx_v)
            handles = []
            for j in range(K):
                handles.append(
                    pltpu.async_copy(
                        table_hbm.at[idx_v.at[j]],
                        rows_v.at[pl.ds(j * SB, SB)],
                        sem,
                    )
                )
            for h in handles:
                h.wait()
            pltpu.sync_copy(rows_v, out_hbm.at[pl.ds(row0 * SB, CH)])
            return carry

        lax.fori_loop(0, n_chunks, body, 0, unroll=False)

    return gather_k


def _proj_body(h_ref, w_ref, b_ref, out_ref):
    h = h_ref[...]
    w = w_ref[...]
    out_ref[...] = lax.dot_general(
        h, w, (((1,), (1,)), ((), ())),
        preferred_element_type=jnp.float32,
    ) + b_ref[...]


def _make_proj(N, D, BLK=8192):
    assert N % BLK == 0
    grid = (N // BLK,)
    return pl.pallas_call(
        _proj_body,
        grid=grid,
        in_specs=[
            pl.BlockSpec((BLK, D), lambda i: (i, 0)),
            pl.BlockSpec((D, D), lambda i: (0, 0)),
            pl.BlockSpec((1, D), lambda i: (0, 0)),
        ],
        out_specs=pl.BlockSpec((BLK, D), lambda i: (i, 0)),
        out_shape=jax.ShapeDtypeStruct((N, D), jnp.float32),
    )


def kernel(x, emb, W, b):
    B, L = x.shape
    V, D = emb.shape
    N = B * L
    idx2d = x.reshape(N // 128, 128)
    gathered = _make_gather(V, D, N)(emb, idx2d)
    out = _make_proj(N, D)(gathered, W, b.reshape(1, D))
    return out.reshape(B, L, D)


# speedup vs baseline: 18.7499x; 1.8384x over previous
import functools

import jax
import jax.numpy as jnp
from jax import lax
from jax.experimental import pallas as pl
from jax.experimental.pallas import tpu as pltpu
from jax.experimental.pallas import tpu_sc as plsc


def _make_gather(V, D, N):
    info = plsc.get_sparse_core_info()
    NC, NS = info.num_cores, info.num_subcores
    NW = NC * NS
    SB = 128
    per_w = N // NW
    assert N % (NW * SB) == 0
    rows_per_w = per_w // SB
    K = 8
    assert rows_per_w % K == 0 and (per_w // SB) % K == 0
    n_chunks = rows_per_w // K
    CH = K * SB

    mesh = plsc.VectorSubcoreMesh(core_axis_name="c", subcore_axis_name="s")

    @functools.partial(
        pl.kernel,
        mesh=mesh,
        out_type=jax.ShapeDtypeStruct((N, D), jnp.float32),
        scratch_types=[
            pltpu.VMEM((K, SB), jnp.int32),
            pltpu.VMEM((CH, D), jnp.float32),
            pltpu.SemaphoreType.DMA,
        ],
        compiler_params=pltpu.CompilerParams(use_tc_tiling_on_sc=False),
    )
    def gather_k(table_hbm, idx_hbm, out_hbm, idx_v, rows_v, sem):
        wid = lax.axis_index("s") * NC + lax.axis_index("c")

        def body(c, carry):
            row0 = wid * rows_per_w + c * K
            pltpu.sync_copy(idx_hbm.at[pl.ds(row0, K)], idx_v)
            handles = []
            for j in range(K):
                handles.append(
                    pltpu.async_copy(
                        table_hbm.at[idx_v.at[j]],
                        rows_v.at[pl.ds(j * SB, SB)],
                        sem,
                    )
                )
            for h in handles:
                h.wait()
            pltpu.sync_copy(rows_v, out_hbm.at[pl.ds(row0 * SB, CH)])
            return carry

        lax.fori_loop(0, n_chunks, body, 0, unroll=False)

    return gather_k


def _proj_body(h_ref, bd_ref, b4_ref, out_ref):
    out_ref[...] = jnp.dot(
        h_ref[...], bd_ref[...], preferred_element_type=jnp.float32
    ) + b4_ref[...]


def _make_proj128(M, BLK=8192):
    assert M % BLK == 0
    grid = (M // BLK,)
    return pl.pallas_call(
        _proj_body,
        grid=grid,
        in_specs=[
            pl.BlockSpec((BLK, 128), lambda i: (i, 0)),
            pl.BlockSpec((128, 128), lambda i: (0, 0)),
            pl.BlockSpec((1, 128), lambda i: (0, 0)),
        ],
        out_specs=pl.BlockSpec((BLK, 128), lambda i: (i, 0)),
        out_shape=jax.ShapeDtypeStruct((M, 128), jnp.float32),
    )


def kernel(x, emb, W, b):
    B, L = x.shape
    V, D = emb.shape
    N = B * L
    R = 128 // D
    idx2d = x.reshape(N // 128, 128)
    gathered = _make_gather(V, D, N)(emb, idx2d)
    g128 = gathered.reshape(N // R, 128)
    bd = jnp.kron(jnp.eye(R, dtype=jnp.float32), W.T)
    b4 = jnp.tile(b, R).reshape(1, 128)
    out128 = _make_proj128(N // R)(g128, bd, b4)
    return out128.reshape(B, L, D)


# speedup vs baseline: 19.9794x; 1.0656x over previous
import functools

import jax
import jax.numpy as jnp
from jax import lax
from jax.experimental import pallas as pl
from jax.experimental.pallas import tpu as pltpu
from jax.experimental.pallas import tpu_sc as plsc


def _make_gather(V, D, N):
    info = plsc.get_sparse_core_info()
    NC, NS = info.num_cores, info.num_subcores
    NW = NC * NS
    SB = 128
    per_w = N // NW
    assert N % (NW * SB) == 0
    rows_per_w = per_w // SB
    K = 8
    assert rows_per_w % K == 0 and (per_w // SB) % K == 0
    n_chunks = rows_per_w // K
    CH = K * SB

    mesh = plsc.VectorSubcoreMesh(core_axis_name="c", subcore_axis_name="s")

    @functools.partial(
        pl.kernel,
        mesh=mesh,
        out_type=jax.ShapeDtypeStruct((N, D), jnp.float32),
        scratch_types=[
            pltpu.VMEM((K, SB), jnp.int32),
            pltpu.VMEM((CH, D), jnp.float32),
            pltpu.SemaphoreType.DMA,
        ],
        compiler_params=pltpu.CompilerParams(use_tc_tiling_on_sc=False),
    )
    def gather_k(table_hbm, idx_hbm, out_hbm, idx_v, rows_v, sem):
        wid = lax.axis_index("s") * NC + lax.axis_index("c")

        def body(c, carry):
            row0 = wid * rows_per_w + c * K
            pltpu.sync_copy(idx_hbm.at[pl.ds(row0, K)], idx_v)
            handles = []
            for j in range(K):
                handles.append(
                    pltpu.async_copy(
                        table_hbm.at[idx_v.at[j]],
                        rows_v.at[pl.ds(j * SB, SB)],
                        sem,
                    )
                )
            for h in handles:
                h.wait()
            pltpu.sync_copy(rows_v, out_hbm.at[pl.ds(row0 * SB, CH)])
            return carry

        lax.fori_loop(0, n_chunks, body, 0, unroll=False)

    return gather_k


def _proj_body(h_ref, bd_ref, b4_ref, out_ref):
    out_ref[...] = jnp.dot(
        h_ref[...], bd_ref[...], preferred_element_type=jnp.float32
    ) + b4_ref[...]


def _make_proj128(M, BLK=8192):
    assert M % BLK == 0
    grid = (M // BLK,)
    return pl.pallas_call(
        _proj_body,
        grid=grid,
        in_specs=[
            pl.BlockSpec((BLK, 128), lambda i: (i, 0)),
            pl.BlockSpec((128, 128), lambda i: (0, 0)),
            pl.BlockSpec((1, 128), lambda i: (0, 0)),
        ],
        out_specs=pl.BlockSpec((BLK, 128), lambda i: (i, 0)),
        out_shape=jax.ShapeDtypeStruct((M, 128), jnp.float32),
    )


def kernel(x, emb, W, b):
    B, L = x.shape
    V, D = emb.shape
    N = B * L
    R = 128 // D
    idx2d = x.T.reshape(N // 128, 128)
    gathered = _make_gather(V, D, N)(emb, idx2d)
    g128 = gathered.reshape(N // R, 128)
    bd = jnp.kron(jnp.eye(R, dtype=jnp.float32), W.T)
    b4 = jnp.tile(b, R).reshape(1, 128)
    out128 = _make_proj128(N // R)(g128, bd, b4)
    return out128.reshape(L, B, D).transpose(1, 0, 2)


# speedup vs baseline: 26.1339x; 1.3080x over previous
import functools

import jax
import jax.numpy as jnp
from jax import lax
from jax.experimental import pallas as pl
from jax.experimental.pallas import tpu as pltpu
from jax.experimental.pallas import tpu_sc as plsc


def _make_gather(V, D, N):
    info = plsc.get_sparse_core_info()
    NC, NS = info.num_cores, info.num_subcores
    NW = NC * NS
    SB = 128
    per_w = N // NW
    assert N % (NW * SB) == 0
    rows_per_w = per_w // SB
    K = 8
    assert rows_per_w % K == 0 and (per_w // SB) % K == 0
    n_chunks = rows_per_w // K
    CH = K * SB

    mesh = plsc.VectorSubcoreMesh(core_axis_name="c", subcore_axis_name="s")

    @functools.partial(
        pl.kernel,
        mesh=mesh,
        out_type=jax.ShapeDtypeStruct((N, D), jnp.float32),
        scratch_types=[
            pltpu.VMEM((K, SB), jnp.int32),
            pltpu.VMEM((CH, D), jnp.float32),
            pltpu.SemaphoreType.DMA,
        ],
        compiler_params=pltpu.CompilerParams(use_tc_tiling_on_sc=False),
    )
    def gather_k(table_hbm, idx_hbm, out_hbm, idx_v, rows_v, sem):
        wid = lax.axis_index("s") * NC + lax.axis_index("c")

        def body(c, carry):
            row0 = wid * rows_per_w + c * K
            pltpu.sync_copy(idx_hbm.at[pl.ds(row0, K)], idx_v)
            handles = []
            for j in range(K):
                handles.append(
                    pltpu.async_copy(
                        table_hbm.at[idx_v.at[j]],
                        rows_v.at[pl.ds(j * SB, SB)],
                        sem,
                    )
                )
            for h in handles:
                h.wait()
            pltpu.sync_copy(rows_v, out_hbm.at[pl.ds(row0 * SB, CH)])
            return carry

        lax.fori_loop(0, n_chunks, body, 0, unroll=False)

    return gather_k


_RB = 1024


def _proj_body_T(h_ref, bd_ref, b_ref, out_ref):
    tt = lax.dot_general(
        bd_ref[...], h_ref[...], (((0,), (1,)), ((), ())),
        preferred_element_type=jnp.float32,
    )
    bcol = b_ref[...]
    for s in range(4):
        out_ref[0, :, s * _RB:(s + 1) * _RB] = tt[32 * s:32 * (s + 1), :] + bcol


def _make_proj_T(L, Bb):
    NJ = Bb // (4 * _RB)
    return pl.pallas_call(
        _proj_body_T,
        grid=(L, NJ),
        in_specs=[
            pl.BlockSpec((_RB, 128), lambda l, j: (l * NJ + j, 0)),
            pl.BlockSpec((128, 128), lambda l, j: (0, 0)),
            pl.BlockSpec((32, 1), lambda l, j: (0, 0)),
        ],
        out_specs=pl.BlockSpec((1, 32, 4 * _RB), lambda l, j: (l, 0, j)),
        out_shape=jax.ShapeDtypeStruct((L, 32, Bb), jnp.float32),
    )


def kernel(x, emb, W, b):
    B, L = x.shape
    V, D = emb.shape
    N = B * L
    idxp = x.T.reshape(L, B // (4 * _RB), 4, _RB)
    idxp = idxp.transpose(0, 1, 3, 2).reshape(N // 128, 128)
    gathered = _make_gather(V, D, N)(emb, idxp)
    g128 = gathered.reshape(N // 4, 128)
    bd = jnp.kron(jnp.eye(4, dtype=jnp.float32), W.T)
    outp = _make_proj_T(L, B)(g128, bd, b.reshape(32, 1))
    return outp.transpose(2, 0, 1)
